# baseline (speedup 1.0000x reference)
import functools

import jax
import jax.numpy as jnp
from jax.experimental import pallas as pl
from jax.experimental.pallas import tpu as pltpu

SD = 16
HALF = SD * 2
FRAMES = HALF // 4
CONN_THRESH = 0.5
K = 100
N = 150
NP = 152
L = 256
KP = 104
NEG = -1e30


def _bcast_phase(a_ref, bc_ref):
    for f in range(FRAMES):
        b0 = 4 * f
        ax1 = a_ref[:, b0 + 0:b0 + 1]
        ay1 = a_ref[:, b0 + 1:b0 + 2]
        ax2 = a_ref[:, b0 + 2:b0 + 3]
        ay2 = a_ref[:, b0 + 3:b0 + 4]
        area_a = jnp.maximum(ax2 - ax1 + 1.0, 0.0) * jnp.maximum(ay2 - ay1 + 1.0, 0.0)
        bc_ref[5 * f + 0] = jnp.broadcast_to(ax1, (NP, L))
        bc_ref[5 * f + 1] = jnp.broadcast_to(ay1, (NP, L))
        bc_ref[5 * f + 2] = jnp.broadcast_to(ax2, (NP, L))
        bc_ref[5 * f + 3] = jnp.broadcast_to(ay2, (NP, L))
        bc_ref[5 * f + 4] = jnp.broadcast_to(area_a, (NP, L))


def _valu_phase(bc_ref, row_ref, sc):
    acc = jnp.zeros((NP, L), jnp.float32)
    for f in range(FRAMES):
        b0 = 4 * f
        ax1 = bc_ref[5 * f + 0]
        ay1 = bc_ref[5 * f + 1]
        ax2 = bc_ref[5 * f + 2]
        ay2 = bc_ref[5 * f + 3]
        area_a = bc_ref[5 * f + 4]
        bx1 = row_ref[b0 + 0:b0 + 1, :]
        by1 = row_ref[b0 + 1:b0 + 2, :]
        bx2 = row_ref[b0 + 2:b0 + 3, :]
        by2 = row_ref[b0 + 3:b0 + 4, :]
        x1 = jnp.maximum(ax1, bx1)
        y1 = jnp.maximum(ay1, by1)
        x2 = jnp.minimum(ax2, bx2)
        y2 = jnp.minimum(ay2, by2)
        iw = jnp.maximum(x2 - x1 + 1.0, 0.0)
        ih = jnp.maximum(y2 - y1 + 1.0, 0.0)
        inter = iw * ih
        area_b = jnp.maximum(bx2 - bx1 + 1.0, 0.0) * jnp.maximum(by2 - by1 + 1.0, 0.0)
        union = (area_a + area_b) - inter
        acc = acc + inter / jnp.maximum(union, 1e-8)
    ov = acc * (1.0 / FRAMES)
    conn = jnp.where(ov > CONN_THRESH, ov, 0.0)
    return jnp.max(sc + conn, axis=0, keepdims=True)


def _body(a_pl, b_tr, act_r, act0_r, out_s, out_i, s_row, bc0, bc1, *, num_t):
    t = pl.program_id(0)
    liota = jax.lax.broadcasted_iota(jnp.int32, (1, L), 1)
    parity = jax.lax.rem(t, 2)

    @pl.when(t == 0)
    def _init():
        s_row[:] = jnp.where(liota < N, act0_r[:], NEG)

    def _stage(bc_rd, bc_wr, par):
        @pl.when(jnp.logical_and(parity == par, t > 0))
        def _valu():
            sc = jnp.transpose(s_row[:], (1, 0))[:NP, :]
            m = _valu_phase(bc_rd, b_tr, sc)
            s_row[:] = jnp.where(liota < N, m + act_r[:], NEG)

        @pl.when(jnp.logical_and(parity == par, t < num_t))
        def _bcast():
            _bcast_phase(a_pl, bc_wr)

    _stage(bc1, bc0, 0)
    _stage(bc0, bc1, 1)

    @pl.when(t == num_t)
    def _topk():
        s = s_row[:]
        rr = jax.lax.broadcasted_iota(jnp.int32, (L, L), 0)
        cc = jax.lax.broadcasted_iota(jnp.int32, (L, L), 1)
        eye = (rr == cc).astype(jnp.float32)
        s_colv = jnp.sum(s * eye, axis=1, keepdims=True)
        gt = (s_colv > s).astype(jnp.int32)
        tie = ((s_colv == s) & (rr < cc)).astype(jnp.int32)
        rank = jnp.sum(gt + tie, axis=0, keepdims=True)
        k_col = jax.lax.broadcasted_iota(jnp.int32, (KP, 1), 0)
        sel = (rank == k_col).astype(jnp.float32)
        out_s[:] = jnp.sum(sel * s, axis=1, keepdims=True)
        lane_f = jax.lax.broadcasted_iota(jnp.int32, (1, L), 1).astype(jnp.float32)
        out_i[:] = jnp.sum(sel * lane_f, axis=1, keepdims=True).astype(jnp.int32)


def kernel(p_tubes, actioness_score):
    t_clips = p_tubes.shape[0]
    num_t = t_clips - 1
    a = p_tubes[:, :, HALF:]
    b = p_tubes[:, :, :HALF]
    a_pl = jnp.pad(a, ((0, 0), (0, NP - N), (0, 0)))
    b_tr = jnp.pad(jnp.transpose(b, (0, 2, 1)), ((0, 0), (0, 0), (0, L - N)))
    act_r = jnp.pad(actioness_score, ((0, 0), (0, L - N)))[:, None, :]

    last_a = num_t - 1
    out_s, out_i = pl.pallas_call(
        functools.partial(_body, num_t=num_t),
        grid=(num_t + 1,),
        in_specs=[
            pl.BlockSpec((None, NP, HALF),
                         lambda t: (jnp.minimum(t, last_a), 0, 0)),
            pl.BlockSpec((None, HALF, L), lambda t: (t, 0, 0)),
            pl.BlockSpec((None, 1, L), lambda t: (t, 0, 0)),
            pl.BlockSpec((None, 1, L), lambda t: (0, 0, 0)),
        ],
        out_specs=[
            pl.BlockSpec((KP, 1), lambda t: (0, 0)),
            pl.BlockSpec((KP, 1), lambda t: (0, 0)),
        ],
        out_shape=[
            jax.ShapeDtypeStruct((KP, 1), jnp.float32),
            jax.ShapeDtypeStruct((KP, 1), jnp.int32),
        ],
        scratch_shapes=[
            pltpu.VMEM((1, L), jnp.float32),
            pltpu.VMEM((FRAMES * 5, NP, L), jnp.float32),
            pltpu.VMEM((FRAMES * 5, NP, L), jnp.float32),
        ],
    )(a_pl, b_tr, act_r, act_r)
    return out_s[:K, 0], out_i[:K, 0]


# speedup vs baseline: 2.0404x; 2.0404x over previous
import functools

import jax
import jax.numpy as jnp
from jax.experimental import pallas as pl
from jax.experimental.pallas import tpu as pltpu

SD = 16
HALF = SD * 2
FRAMES = HALF // 4
CONN_THRESH = 0.5
K = 100
N = 150
NP = 152
L = 256
KP = 104
NEG = -1e30


def _conn(col_ref, row_ref):
    acc = jnp.zeros((NP, L), jnp.float32)
    for f in range(FRAMES):
        b0 = 4 * f
        ax1 = col_ref[:, b0 + 0:b0 + 1]
        ay1 = col_ref[:, b0 + 1:b0 + 2]
        ax2 = col_ref[:, b0 + 2:b0 + 3]
        ay2 = col_ref[:, b0 + 3:b0 + 4]
        bx1 = row_ref[b0 + 0:b0 + 1, :]
        by1 = row_ref[b0 + 1:b0 + 2, :]
        bx2 = row_ref[b0 + 2:b0 + 3, :]
        by2 = row_ref[b0 + 3:b0 + 4, :]
        x1 = jnp.maximum(ax1, bx1)
        y1 = jnp.maximum(ay1, by1)
        x2 = jnp.minimum(ax2, bx2)
        y2 = jnp.minimum(ay2, by2)
        iw = jnp.maximum(x2 - x1 + 1.0, 0.0)
        ih = jnp.maximum(y2 - y1 + 1.0, 0.0)
        inter = iw * ih
        area_a = jnp.maximum(ax2 - ax1 + 1.0, 0.0) * jnp.maximum(ay2 - ay1 + 1.0, 0.0)
        area_b = jnp.maximum(bx2 - bx1 + 1.0, 0.0) * jnp.maximum(by2 - by1 + 1.0, 0.0)
        union = (area_a + area_b) - inter
        acc = acc + inter / jnp.maximum(union, 1e-8)
    ov = acc * (1.0 / FRAMES)
    return jnp.where(ov > CONN_THRESH, ov, 0.0)


def _body(a_pl, b_tr, act_r, act0_r, out_s, out_i, s_row, *, num_t):
    t = pl.program_id(0)
    liota = jax.lax.broadcasted_iota(jnp.int32, (1, L), 1)

    @pl.when(t == 0)
    def _init():
        s_row[:] = jnp.where(liota < N, act0_r[:], NEG)

    sc = jnp.transpose(s_row[:], (1, 0))[:NP, :]
    conn = _conn(a_pl, b_tr)
    m = jnp.max(sc + conn, axis=0, keepdims=True)
    s_row[:] = jnp.where(liota < N, m + act_r[:], NEG)

    @pl.when(t == num_t - 1)
    def _topk():
        s = s_row[:]
        rr = jax.lax.broadcasted_iota(jnp.int32, (L, L), 0)
        cc = jax.lax.broadcasted_iota(jnp.int32, (L, L), 1)
        eye = (rr == cc).astype(jnp.float32)
        s_colv = jnp.sum(s * eye, axis=1, keepdims=True)
        gt = (s_colv > s).astype(jnp.int32)
        tie = ((s_colv == s) & (rr < cc)).astype(jnp.int32)
        rank = jnp.sum(gt + tie, axis=0, keepdims=True)
        k_col = jax.lax.broadcasted_iota(jnp.int32, (KP, 1), 0)
        sel = (rank == k_col).astype(jnp.float32)
        out_s[:] = jnp.sum(sel * s, axis=1, keepdims=True)
        lane_f = jax.lax.broadcasted_iota(jnp.int32, (1, L), 1).astype(jnp.float32)
        out_i[:] = jnp.sum(sel * lane_f, axis=1, keepdims=True).astype(jnp.int32)


def kernel(p_tubes, actioness_score):
    t_clips = p_tubes.shape[0]
    num_t = t_clips - 1
    a = p_tubes[:, :, HALF:]
    b = p_tubes[:, :, :HALF]
    a_pl = jnp.pad(a, ((0, 0), (0, NP - N), (0, 0)))
    b_tr = jnp.pad(jnp.transpose(b, (0, 2, 1)), ((0, 0), (0, 0), (0, L - N)))
    act_r = jnp.pad(actioness_score, ((0, 0), (0, L - N)))[:, None, :]

    out_s, out_i = pl.pallas_call(
        functools.partial(_body, num_t=num_t),
        grid=(num_t,),
        in_specs=[
            pl.BlockSpec((None, NP, HALF), lambda t: (t, 0, 0)),
            pl.BlockSpec((None, HALF, L), lambda t: (t + 1, 0, 0)),
            pl.BlockSpec((None, 1, L), lambda t: (t + 1, 0, 0)),
            pl.BlockSpec((None, 1, L), lambda t: (0, 0, 0)),
        ],
        out_specs=[
            pl.BlockSpec((KP, 1), lambda t: (0, 0)),
            pl.BlockSpec((KP, 1), lambda t: (0, 0)),
        ],
        out_shape=[
            jax.ShapeDtypeStruct((KP, 1), jnp.float32),
            jax.ShapeDtypeStruct((KP, 1), jnp.int32),
        ],
        scratch_shapes=[
            pltpu.VMEM((1, L), jnp.float32),
        ],
    )(a_pl, b_tr, act_r, act_r)
    return out_s[:K, 0], out_i[:K, 0]


# speedup vs baseline: 2.0519x; 1.0057x over previous
import functools

import jax
import jax.numpy as jnp
from jax.experimental import pallas as pl
from jax.experimental.pallas import tpu as pltpu

SD = 16
HALF = SD * 2
FRAMES = HALF // 4
CONN_THRESH = 0.5
K = 100
N = 150
NP = 152
L = 256
KP = 104
NEG = -1e30


def _conn(col_ref, row_ref):
    acc = jnp.zeros((NP, L), jnp.float32)
    for f in range(FRAMES):
        b0 = 4 * f
        ax1 = col_ref[:, b0 + 0:b0 + 1]
        ay1 = col_ref[:, b0 + 1:b0 + 2]
        ax2 = col_ref[:, b0 + 2:b0 + 3]
        ay2 = col_ref[:, b0 + 3:b0 + 4]
        bx1 = row_ref[b0 + 0:b0 + 1, :]
        by1 = row_ref[b0 + 1:b0 + 2, :]
        bx2 = row_ref[b0 + 2:b0 + 3, :]
        by2 = row_ref[b0 + 3:b0 + 4, :]
        x1 = jnp.maximum(ax1, bx1)
        y1 = jnp.maximum(ay1, by1)
        x2 = jnp.minimum(ax2, bx2)
        y2 = jnp.minimum(ay2, by2)
        iw = jnp.maximum(x2 - x1 + 1.0, 0.0)
        ih = jnp.maximum(y2 - y1 + 1.0, 0.0)
        inter = iw * ih
        area_a = jnp.maximum(ax2 - ax1 + 1.0, 0.0) * jnp.maximum(ay2 - ay1 + 1.0, 0.0)
        area_b = jnp.maximum(bx2 - bx1 + 1.0, 0.0) * jnp.maximum(by2 - by1 + 1.0, 0.0)
        union = (area_a + area_b) - inter
        acc = acc + inter / jnp.maximum(union, 1e-8)
    ov = acc * (1.0 / FRAMES)
    return jnp.where(ov > CONN_THRESH, ov, 0.0)


def _body(a0_pl, a1_pl, b1_tr, b2_tr, act1_r, act2_r, act0_r,
          out_s, out_i, s_row, *, num_t, num_g):
    g = pl.program_id(0)
    liota = jax.lax.broadcasted_iota(jnp.int32, (1, L), 1)

    init_row = jnp.where(liota < N, act0_r[:], NEG)
    s0 = jnp.where(g == 0, init_row, s_row[:])

    sc0 = jnp.transpose(s0, (1, 0))[:NP, :]
    conn0 = _conn(a0_pl, b1_tr)
    m0 = jnp.max(sc0 + conn0, axis=0, keepdims=True)
    s1 = jnp.where(liota < N, m0 + act1_r[:], NEG)

    sc1 = jnp.transpose(s1, (1, 0))[:NP, :]
    conn1 = _conn(a1_pl, b2_tr)
    m1 = jnp.max(sc1 + conn1, axis=0, keepdims=True)
    s2 = jnp.where(liota < N, m1 + act2_r[:], NEG)

    s_row[:] = jnp.where(2 * g + 1 < num_t, s2, s1)

    @pl.when(g == num_g - 1)
    def _topk():
        s = s_row[:]
        rr = jax.lax.broadcasted_iota(jnp.int32, (L, L), 0)
        cc = jax.lax.broadcasted_iota(jnp.int32, (L, L), 1)
        eye = (rr == cc).astype(jnp.float32)
        s_colv = jnp.sum(s * eye, axis=1, keepdims=True)
        gt = (s_colv > s).astype(jnp.int32)
        tie = ((s_colv == s) & (rr < cc)).astype(jnp.int32)
        rank = jnp.sum(gt + tie, axis=0, keepdims=True)
        k_col = jax.lax.broadcasted_iota(jnp.int32, (KP, 1), 0)
        sel = (rank == k_col).astype(jnp.float32)
        out_s[:] = jnp.sum(sel * s, axis=1, keepdims=True)
        lane_f = jax.lax.broadcasted_iota(jnp.int32, (1, L), 1).astype(jnp.float32)
        out_i[:] = jnp.sum(sel * lane_f, axis=1, keepdims=True).astype(jnp.int32)


def kernel(p_tubes, actioness_score):
    t_clips = p_tubes.shape[0]
    num_t = t_clips - 1
    a = p_tubes[:, :, HALF:]
    b = p_tubes[:, :, :HALF]
    a_pl = jnp.pad(a, ((0, 0), (0, NP - N), (0, 0)))
    b_tr = jnp.pad(jnp.transpose(b, (0, 2, 1)), ((0, 0), (0, 0), (0, L - N)))
    act_r = jnp.pad(actioness_score, ((0, 0), (0, L - N)))[:, None, :]

    num_g = (num_t + 1) // 2
    last = t_clips - 1
    out_s, out_i = pl.pallas_call(
        functools.partial(_body, num_t=num_t, num_g=num_g),
        grid=(num_g,),
        in_specs=[
            pl.BlockSpec((None, NP, HALF), lambda g: (2 * g, 0, 0)),
            pl.BlockSpec((None, NP, HALF),
                         lambda g: (jnp.minimum(2 * g + 1, last - 1), 0, 0)),
            pl.BlockSpec((None, HALF, L), lambda g: (2 * g + 1, 0, 0)),
            pl.BlockSpec((None, HALF, L),
                         lambda g: (jnp.minimum(2 * g + 2, last), 0, 0)),
            pl.BlockSpec((None, 1, L), lambda g: (2 * g + 1, 0, 0)),
            pl.BlockSpec((None, 1, L),
                         lambda g: (jnp.minimum(2 * g + 2, last), 0, 0)),
            pl.BlockSpec((None, 1, L), lambda g: (0, 0, 0)),
        ],
        out_specs=[
            pl.BlockSpec((KP, 1), lambda g: (0, 0)),
            pl.BlockSpec((KP, 1), lambda g: (0, 0)),
        ],
        out_shape=[
            jax.ShapeDtypeStruct((KP, 1), jnp.float32),
            jax.ShapeDtypeStruct((KP, 1), jnp.int32),
        ],
        scratch_shapes=[
            pltpu.VMEM((1, L), jnp.float32),
        ],
    )(a_pl, a_pl, b_tr, b_tr, act_r, act_r, act_r)
    return out_s[:K, 0], out_i[:K, 0]
